# baseline (speedup 1.0000x reference)
import jax
import jax.numpy as jnp
from jax.experimental import pallas as pl
from jax.experimental.pallas import tpu as pltpu

GROWTH = 32
CMID = 128
BLOCK_CONFIG = (6, 12, 24, 16)
C0S = (64, 128, 256, 512)
HS = (16, 8, 4, 2)
NC = 8


WPADS = (32, 16, 16, 16)


def _geom(h, wpad):
    hp = h + 2
    rows_p = NC * hp * wpad
    base = wpad + 16
    return hp, rows_p, base


def _interior_mask(h, wpad):
    hp = h + 2
    yi = jax.lax.broadcasted_iota(jnp.int32, (NC, hp, wpad, 1), 1)
    xi = jax.lax.broadcasted_iota(jnp.int32, (NC, hp, wpad, 1), 2)
    ok = (yi >= 1) & (yi <= h) & (xi >= 1) & (xi <= h)
    return ok.reshape(NC * hp * wpad, 1)


def _pad_hw(y4, wpad):
    h = y4.shape[1]
    return jnp.pad(y4, ((0, 0), (1, 1), (1, wpad - h - 1), (0, 0)))


def _maxpool3x3s2(y):
    n, hh, _, c = y.shape
    ho = hh // 2
    pr = y.reshape(n, hh, ho, 2, c)
    e = jnp.maximum(pr[:, :, :, 0, :], pr[:, :, :, 1, :])
    o = jnp.concatenate(
        [jnp.zeros((n, hh, 1, c), y.dtype), pr[:, :, :ho - 1, 1, :]], axis=2)
    mw = jnp.maximum(e, o)
    pr2 = mw.reshape(n, ho, 2, ho, c)
    e2 = jnp.maximum(pr2[:, :, 0, :, :], pr2[:, :, 1, :, :])
    o2 = jnp.concatenate(
        [jnp.zeros((n, 1, ho, c), y.dtype), pr2[:, :ho - 1, 1, :, :]], axis=1)
    return jnp.maximum(e2, o2)


def _dense_layer(slab_ref, tbuf_ref, s1, t1, w1, s2, t2, w2, mask, h, wpad,
                 cin):
    hp, rows_p, base = _geom(h, wpad)
    a = jnp.maximum(slab_ref[:, :cin] * s1 + t1, 0.0)
    t = jnp.dot(a.astype(jnp.bfloat16), w1, preferred_element_type=jnp.float32)
    t = jnp.maximum(t * s2 + t2, 0.0)
    t = jnp.where(mask, t, 0.0).astype(jnp.bfloat16)
    tbuf_ref[pl.ds(base, rows_p), :] = t
    g = [None, None, None]
    for di in range(3):
        v = tbuf_ref[pl.ds(di * wpad, rows_p + 32), :]
        for dj in range(3):
            k = di * 3 + dj
            d = jnp.dot(v, w2[k * CMID:(k + 1) * CMID, :],
                        preferred_element_type=jnp.float32)
            g[dj] = d if g[dj] is None else g[dj] + d
    acc = (g[0][15:15 + rows_p] + g[1][16:16 + rows_p]
           + g[2][17:17 + rows_p])
    slab_ref[:, cin:cin + GROWTH] = acc


def _forward(p_ref, w0_ref, s0_ref, t0_ref, *rest):
    blocks = [rest[6 * i:6 * i + 6] for i in range(4)]
    trans = [rest[24 + 3 * i:24 + 3 * i + 3] for i in range(3)]
    s5_ref, t5_ref, wd_ref, bd_ref, o_ref = rest[33:38]
    slabs = list(rest[38:42])
    tbufs = list(rest[42:46])

    for tb in tbufs:
        tb[...] = jnp.zeros_like(tb)

    y = jax.lax.dot_general(p_ref[...], w0_ref[...],
                            (((0,), (0,)), ((), ())),
                            preferred_element_type=jnp.float32)
    y = jnp.maximum(y * s0_ref[...] + t0_ref[...], 0.0)
    y4 = _maxpool3x3s2(y.reshape(NC, 32, 32, 64))
    slabs[0][:, :C0S[0]] = _pad_hw(y4, WPADS[0]).reshape(-1, C0S[0])

    for b in range(4):
        s1_ref, t1_ref, w1_ref, s2_ref, t2_ref, w2_ref = blocks[b]
        h, wpad, c0, num_layers = HS[b], WPADS[b], C0S[b], BLOCK_CONFIG[b]
        mask = _interior_mask(h, wpad)
        for l in range(num_layers):
            cin = c0 + l * GROWTH
            _dense_layer(slabs[b], tbufs[b],
                         s1_ref[l, :, :cin], t1_ref[l, :, :cin],
                         w1_ref[l, :cin, :], s2_ref[l], t2_ref[l], w2_ref[l],
                         mask, h, wpad, cin)
        if b < 3:
            ts_ref, tt_ref, tw_ref = trans[b]
            cfull = c0 + num_layers * GROWTH
            hp = h + 2
            a = jnp.maximum(slabs[b][...] * ts_ref[...] + tt_ref[...], 0.0)
            a4 = a.reshape(NC, hp, wpad, cfull)[:, 1:h + 1, 1:h + 1, :]
            h1 = h // 2
            pr = a4.reshape(NC, h1, 2, h1, 2, cfull)
            pooled = (pr[:, :, 0, :, 0, :] + pr[:, :, 1, :, 0, :]
                      + pr[:, :, 0, :, 1, :] + pr[:, :, 1, :, 1, :]) * 0.25
            ynext = jnp.dot(pooled.reshape(NC * h1 * h1, cfull).astype(jnp.bfloat16),
                            tw_ref[...], preferred_element_type=jnp.float32)
            cnext = cfull // 2
            slabs[b + 1][:, :cnext] = _pad_hw(
                ynext.reshape(NC, h1, h1, cnext), WPADS[b + 1]).reshape(-1, cnext)

    cfin = C0S[3] + BLOCK_CONFIG[3] * GROWTH
    a = jnp.maximum(slabs[3][...] * s5_ref[0] + t5_ref[0], 0.0)
    a4 = a.reshape(NC, 4, WPADS[3], cfin)[:, 1:3, 1:3, :]
    pooled = (a4[:, 0, 0, :] + a4[:, 0, 1, :]
              + a4[:, 1, 0, :] + a4[:, 1, 1, :]) * 0.25
    logits = jnp.dot(pooled.astype(jnp.bfloat16), wd_ref[...],
                     preferred_element_type=jnp.float32) + bd_ref[...]
    o_ref[0] = logits


def _full_spec(a):
    shape = tuple(a.shape)
    zeros = (0,) * len(shape)
    return pl.BlockSpec(shape, lambda b, _z=zeros: _z)


def kernel(conv0_w, s0, t0,
           block0_s1, block0_t1, block0_w1, block0_s2, block0_t2, block0_w2,
           block1_s1, block1_t1, block1_w1, block1_s2, block1_t2, block1_w2,
           block2_s1, block2_t1, block2_w1, block2_s2, block2_t2, block2_w2,
           block3_s1, block3_t1, block3_w1, block3_s2, block3_t2, block3_w2,
           trans0_s, trans0_t, trans0_w,
           trans1_s, trans1_t, trans1_w,
           trans2_s, trans2_t, trans2_w,
           s5, t5, wd, bd, x):
    n = x.shape[0]
    xp = jnp.pad(x, ((0, 0), (0, 0), (3, 3), (3, 3)))
    xp = jnp.transpose(xp, (1, 0, 2, 3)).astype(jnp.bfloat16)
    par = [[xp[:, :, pi::2, pj::2] for pj in range(2)] for pi in range(2)]
    rows = [par[i % 2][j % 2][:, :, i // 2:i // 2 + 32, j // 2:j // 2 + 32]
            .reshape(3, n * 32 * 32)
            for i in range(7) for j in range(7)]
    patches = jnp.concatenate(rows, axis=0)

    weights = (conv0_w, s0, t0,
               block0_s1, block0_t1, block0_w1, block0_s2, block0_t2, block0_w2,
               block1_s1, block1_t1, block1_w1, block1_s2, block1_t2, block1_w2,
               block2_s1, block2_t1, block2_w1, block2_s2, block2_t2, block2_w2,
               block3_s1, block3_t1, block3_w1, block3_s2, block3_t2, block3_w2,
               trans0_s, trans0_t, trans0_w,
               trans1_s, trans1_t, trans1_w,
               trans2_s, trans2_t, trans2_w,
               s5, t5, wd, bd)

    scratch = []
    for b in range(4):
        _, rows_p, _ = _geom(HS[b], WPADS[b])
        cfull = C0S[b] + BLOCK_CONFIG[b] * GROWTH
        scratch.append(pltpu.VMEM((rows_p, cfull), jnp.float32))
    for b in range(4):
        _, rows_p, _ = _geom(HS[b], WPADS[b])
        scratch.append(pltpu.VMEM((rows_p + 2 * WPADS[b] + 32, CMID),
                                  jnp.bfloat16))

    out = pl.pallas_call(
        _forward,
        out_shape=jax.ShapeDtypeStruct((1, NC, 4), jnp.float32),
        grid=(1,),
        in_specs=[pl.BlockSpec((147, NC * 32 * 32), lambda b: (0, 0))]
        + [_full_spec(a) for a in weights],
        out_specs=pl.BlockSpec((1, NC, 4), lambda b: (b, 0, 0)),
        scratch_shapes=scratch,
        compiler_params=pltpu.CompilerParams(
            dimension_semantics=("arbitrary",)),
    )(patches, *weights)
    return out.reshape(n, 4)


# speedup vs baseline: 1.0928x; 1.0928x over previous
import jax
import jax.numpy as jnp
from jax.experimental import pallas as pl
from jax.experimental.pallas import tpu as pltpu

GROWTH = 32
CMID = 128
BLOCK_CONFIG = (6, 12, 24, 16)
C0S = (64, 128, 256, 512)
HS = (16, 8, 4, 2)
NC = 8


WPADS = (32, 16, 16, 16)


def _geom(h, wpad):
    hp = h + 2
    rows_p = NC * hp * wpad
    base = wpad + 16
    return hp, rows_p, base


def _interior_mask(h, wpad):
    hp = h + 2
    yi = jax.lax.broadcasted_iota(jnp.int32, (NC, hp, wpad, 1), 1)
    xi = jax.lax.broadcasted_iota(jnp.int32, (NC, hp, wpad, 1), 2)
    ok = (yi >= 1) & (yi <= h) & (xi >= 1) & (xi <= h)
    return ok.reshape(NC * hp * wpad, 1)


def _pad_hw(y4, wpad):
    h = y4.shape[1]
    return jnp.pad(y4, ((0, 0), (1, 1), (1, wpad - h - 1), (0, 0)))


def _maxpool3x3s2(y):
    n, hh, _, c = y.shape
    ho = hh // 2
    pr = y.reshape(n, hh, ho, 2, c)
    e = jnp.maximum(pr[:, :, :, 0, :], pr[:, :, :, 1, :])
    o = jnp.concatenate(
        [jnp.zeros((n, hh, 1, c), y.dtype), pr[:, :, :ho - 1, 1, :]], axis=2)
    mw = jnp.maximum(e, o)
    pr2 = mw.reshape(n, ho, 2, ho, c)
    e2 = jnp.maximum(pr2[:, :, 0, :, :], pr2[:, :, 1, :, :])
    o2 = jnp.concatenate(
        [jnp.zeros((n, 1, ho, c), y.dtype), pr2[:, :ho - 1, 1, :, :]], axis=1)
    return jnp.maximum(e2, o2)


def _dense_layer(slab_ref, tbuf_ref, s1, t1, w1, s2, t2, w2, mask, h, wpad,
                 cin):
    hp, rows_p, base = _geom(h, wpad)
    a = jnp.maximum(slab_ref[:, :cin] * s1 + t1, 0.0)
    t = jnp.dot(a.astype(jnp.bfloat16), w1, preferred_element_type=jnp.float32)
    t = jnp.maximum(t * s2 + t2, 0.0)
    t = jnp.where(mask, t, 0.0).astype(jnp.bfloat16)
    tbuf_ref[pl.ds(base, rows_p), :] = t
    g = [None, None, None]
    for di in range(3):
        v = tbuf_ref[pl.ds(di * wpad, rows_p + 32), :]
        for dj in range(3):
            k = di * 3 + dj
            d = jnp.dot(v, w2[k * CMID:(k + 1) * CMID, :],
                        preferred_element_type=jnp.float32)
            g[dj] = d if g[dj] is None else g[dj] + d
    acc = (g[0][15:15 + rows_p] + g[1][16:16 + rows_p]
           + g[2][17:17 + rows_p])
    slab_ref[:, cin:cin + GROWTH] = acc


def _forward(p_ref, w0_ref, s0_ref, t0_ref, *rest):
    blocks = [rest[6 * i:6 * i + 6] for i in range(4)]
    trans = [rest[24 + 3 * i:24 + 3 * i + 3] for i in range(3)]
    s5_ref, t5_ref, wd_ref, bd_ref, o_ref = rest[33:38]
    slabs = list(rest[38:42])
    tbufs = list(rest[42:46])

    for tb in tbufs:
        tb[...] = jnp.zeros_like(tb)

    y = jax.lax.dot_general(p_ref[...], w0_ref[...],
                            (((0,), (0,)), ((), ())),
                            preferred_element_type=jnp.float32)
    y = jnp.maximum(y * s0_ref[...] + t0_ref[...], 0.0)
    y4 = _maxpool3x3s2(y.reshape(NC, 32, 32, 64))
    slabs[0][:, :C0S[0]] = _pad_hw(y4, WPADS[0]).reshape(-1, C0S[0])

    for b in range(4):
        s1_ref, t1_ref, w1_ref, s2_ref, t2_ref, w2_ref = blocks[b]
        h, wpad, c0, num_layers = HS[b], WPADS[b], C0S[b], BLOCK_CONFIG[b]
        mask = _interior_mask(h, wpad)
        for l in range(num_layers):
            cin = c0 + l * GROWTH
            _dense_layer(slabs[b], tbufs[b],
                         s1_ref[l, :, :cin], t1_ref[l, :, :cin],
                         w1_ref[l, :cin, :], s2_ref[l], t2_ref[l], w2_ref[l],
                         mask, h, wpad, cin)
        if b < 3:
            ts_ref, tt_ref, tw_ref = trans[b]
            cfull = c0 + num_layers * GROWTH
            hp = h + 2
            a = jnp.maximum(slabs[b][...] * ts_ref[...] + tt_ref[...], 0.0)
            a4 = a.reshape(NC, hp, wpad, cfull)[:, 1:h + 1, 1:h + 1, :]
            h1 = h // 2
            pr = a4.reshape(NC, h1, 2, h1, 2, cfull)
            pooled = (pr[:, :, 0, :, 0, :] + pr[:, :, 1, :, 0, :]
                      + pr[:, :, 0, :, 1, :] + pr[:, :, 1, :, 1, :]) * 0.25
            ynext = jnp.dot(pooled.reshape(NC * h1 * h1, cfull).astype(jnp.bfloat16),
                            tw_ref[...], preferred_element_type=jnp.float32)
            cnext = cfull // 2
            slabs[b + 1][:, :cnext] = _pad_hw(
                ynext.reshape(NC, h1, h1, cnext), WPADS[b + 1]).reshape(-1, cnext)

    cfin = C0S[3] + BLOCK_CONFIG[3] * GROWTH
    a = jnp.maximum(slabs[3][...] * s5_ref[0] + t5_ref[0], 0.0)
    a4 = a.reshape(NC, 4, WPADS[3], cfin)[:, 1:3, 1:3, :]
    pooled = (a4[:, 0, 0, :] + a4[:, 0, 1, :]
              + a4[:, 1, 0, :] + a4[:, 1, 1, :]) * 0.25
    logits = jnp.dot(pooled.astype(jnp.bfloat16), wd_ref[...],
                     preferred_element_type=jnp.float32) + bd_ref[...]
    o_ref[0] = logits


def _full_spec(a):
    shape = tuple(a.shape)
    zeros = (0,) * len(shape)
    return pl.BlockSpec(shape, lambda b, _z=zeros: _z)


def kernel(conv0_w, s0, t0,
           block0_s1, block0_t1, block0_w1, block0_s2, block0_t2, block0_w2,
           block1_s1, block1_t1, block1_w1, block1_s2, block1_t2, block1_w2,
           block2_s1, block2_t1, block2_w1, block2_s2, block2_t2, block2_w2,
           block3_s1, block3_t1, block3_w1, block3_s2, block3_t2, block3_w2,
           trans0_s, trans0_t, trans0_w,
           trans1_s, trans1_t, trans1_w,
           trans2_s, trans2_t, trans2_w,
           s5, t5, wd, bd, x):
    n = x.shape[0]
    xp = jnp.pad(x, ((0, 0), (0, 0), (3, 3), (3, 3)))
    xp = xp.astype(jnp.bfloat16)
    par = [[xp[:, :, pi::2, pj::2] for pj in range(2)] for pi in range(2)]
    rows = [par[i % 2][j % 2][:, c, i // 2:i // 2 + 32, j // 2:j // 2 + 32]
            .reshape(1, n * 32 * 32)
            for i in range(7) for j in range(7) for c in range(3)]
    patches = jnp.concatenate(rows, axis=0)

    weights = (conv0_w, s0, t0,
               block0_s1, block0_t1, block0_w1, block0_s2, block0_t2, block0_w2,
               block1_s1, block1_t1, block1_w1, block1_s2, block1_t2, block1_w2,
               block2_s1, block2_t1, block2_w1, block2_s2, block2_t2, block2_w2,
               block3_s1, block3_t1, block3_w1, block3_s2, block3_t2, block3_w2,
               trans0_s, trans0_t, trans0_w,
               trans1_s, trans1_t, trans1_w,
               trans2_s, trans2_t, trans2_w,
               s5, t5, wd, bd)

    scratch = []
    for b in range(4):
        _, rows_p, _ = _geom(HS[b], WPADS[b])
        cfull = C0S[b] + BLOCK_CONFIG[b] * GROWTH
        scratch.append(pltpu.VMEM((rows_p, cfull), jnp.float32))
    for b in range(4):
        _, rows_p, _ = _geom(HS[b], WPADS[b])
        scratch.append(pltpu.VMEM((rows_p + 2 * WPADS[b] + 32, CMID),
                                  jnp.bfloat16))

    out = pl.pallas_call(
        _forward,
        out_shape=jax.ShapeDtypeStruct((1, NC, 4), jnp.float32),
        grid=(1,),
        in_specs=[pl.BlockSpec((147, NC * 32 * 32), lambda b: (0, 0))]
        + [_full_spec(a) for a in weights],
        out_specs=pl.BlockSpec((1, NC, 4), lambda b: (b, 0, 0)),
        scratch_shapes=scratch,
        compiler_params=pltpu.CompilerParams(
            dimension_semantics=("arbitrary",)),
    )(patches, *weights)
    return out.reshape(n, 4)
